# baseline (speedup 1.0000x reference)
import functools

import jax
import jax.numpy as jnp
from jax import lax
from jax.experimental import pallas as pl
from jax.experimental.pallas import tpu as pltpu

_EPS = 1e-8
_MATCH_THRESH = 0.999
_BB = 32

_NT = (((1,), (1,)), ((), ()))


def _bias_kernel(hypo_ref, prem_ref, w_ref, b_ref, out_ref, *, n_pairs, n_h):
    d = hypo_ref.shape[2]
    inv_h = 1.0 / float(n_h)
    ones_row = jnp.ones((1, d), jnp.float32)
    rows = []
    for i in range(n_pairs):
        h = hypo_ref[i]
        p = prem_ref[i]
        ssq_h = lax.dot_general(h * h, ones_row, _NT,
                                preferred_element_type=jnp.float32)
        ssq_p = lax.dot_general(ones_row, p * p, _NT,
                                preferred_element_type=jnp.float32)
        ih = 1.0 / (jnp.sqrt(ssq_h) + _EPS)
        ip = 1.0 / (jnp.sqrt(ssq_p) + _EPS)
        g = lax.dot_general(h, p, _NT,
                            preferred_element_type=jnp.float32)
        c = g * ih * ip
        mx = jnp.max(c, axis=1, keepdims=True)
        s_mx = jnp.sum(mx, axis=0, keepdims=True)
        mn = jnp.min(mx, axis=0, keepdims=True)
        m = jnp.where(c > _MATCH_THRESH, 1.0, 0.0)
        cnt = jnp.sum(jnp.sum(m, axis=1, keepdims=True), axis=0,
                      keepdims=True)
        f0 = (mn > _MATCH_THRESH).astype(jnp.float32)
        f1 = (cnt == float(n_h)).astype(jnp.float32)
        f2 = cnt * inv_h
        f3 = 1.0 - s_mx * inv_h
        f4 = 1.0 - mn
        rows.append(jnp.concatenate([f0, f1, f2, f3, f4], axis=1))
    feats = jnp.concatenate(rows, axis=0)
    out = lax.dot_general(feats, w_ref[:], _NT,
                          preferred_element_type=jnp.float32)
    out_ref[:] = out + b_ref[:]


@jax.jit
def kernel(hypo, prem, W, b):
    B, H, D = hypo.shape
    P = prem.shape[1]
    bb = _BB
    grid = (B // bb,)
    b2 = b.reshape(1, 3)
    return pl.pallas_call(
        functools.partial(_bias_kernel, n_pairs=bb, n_h=H),
        grid=grid,
        in_specs=[
            pl.BlockSpec((bb, H, D), lambda i: (i, 0, 0)),
            pl.BlockSpec((bb, P, D), lambda i: (i, 0, 0)),
            pl.BlockSpec((3, 5), lambda i: (0, 0)),
            pl.BlockSpec((1, 3), lambda i: (0, 0)),
        ],
        out_specs=pl.BlockSpec((bb, 3), lambda i: (i, 0)),
        out_shape=jax.ShapeDtypeStruct((B, 3), jnp.float32),
        compiler_params=pltpu.CompilerParams(
            dimension_semantics=("parallel",),
        ),
    )(hypo, prem, W, b2)


# speedup vs baseline: 1.0680x; 1.0680x over previous
import functools

import jax
import jax.numpy as jnp
from jax import lax
from jax.experimental import pallas as pl
from jax.experimental.pallas import tpu as pltpu

_EPS = 1e-8
_MATCH_THRESH = 0.999
_BB = 64

_NT = (((1,), (1,)), ((), ()))


def _bias_kernel(hypo_ref, prem_ref, w_ref, b_ref, out_ref, *, n_pairs, n_h):
    d = hypo_ref.shape[2]
    inv_h = 1.0 / float(n_h)
    ones_row = jnp.ones((1, d), jnp.float32)
    rows = []
    for i in range(n_pairs):
        h = hypo_ref[i]
        p = prem_ref[i]
        ssq_h = lax.dot_general(h * h, ones_row, _NT,
                                preferred_element_type=jnp.float32)
        ssq_p = lax.dot_general(ones_row, p * p, _NT,
                                preferred_element_type=jnp.float32)
        ih = 1.0 / (jnp.sqrt(ssq_h) + _EPS)
        ip = 1.0 / (jnp.sqrt(ssq_p) + _EPS)
        g = lax.dot_general(h, p, _NT,
                            preferred_element_type=jnp.float32)
        c = g * ih * ip
        mx = jnp.max(c, axis=1, keepdims=True)
        s_mx = jnp.sum(mx, axis=0, keepdims=True)
        mn = jnp.min(mx, axis=0, keepdims=True)
        m = jnp.where(c > _MATCH_THRESH, 1.0, 0.0)
        cnt = jnp.sum(jnp.sum(m, axis=1, keepdims=True), axis=0,
                      keepdims=True)
        f0 = (mn > _MATCH_THRESH).astype(jnp.float32)
        f1 = (cnt == float(n_h)).astype(jnp.float32)
        f2 = cnt * inv_h
        f3 = 1.0 - s_mx * inv_h
        f4 = 1.0 - mn
        rows.append(jnp.concatenate([f0, f1, f2, f3, f4], axis=1))
    feats = jnp.concatenate(rows, axis=0)
    out = lax.dot_general(feats, w_ref[:], _NT,
                          preferred_element_type=jnp.float32)
    out_ref[:] = out + b_ref[:]


@jax.jit
def kernel(hypo, prem, W, b):
    B, H, D = hypo.shape
    P = prem.shape[1]
    bb = _BB
    grid = (B // bb,)
    b2 = b.reshape(1, 3)
    return pl.pallas_call(
        functools.partial(_bias_kernel, n_pairs=bb, n_h=H),
        grid=grid,
        in_specs=[
            pl.BlockSpec((bb, H, D), lambda i: (i, 0, 0)),
            pl.BlockSpec((bb, P, D), lambda i: (i, 0, 0)),
            pl.BlockSpec((3, 5), lambda i: (0, 0)),
            pl.BlockSpec((1, 3), lambda i: (0, 0)),
        ],
        out_specs=pl.BlockSpec((bb, 3), lambda i: (i, 0)),
        out_shape=jax.ShapeDtypeStruct((B, 3), jnp.float32),
        compiler_params=pltpu.CompilerParams(
            dimension_semantics=("parallel",),
        ),
    )(hypo, prem, W, b2)


# speedup vs baseline: 1.1088x; 1.0382x over previous
import functools

import jax
import jax.numpy as jnp
from jax import lax
from jax.experimental import pallas as pl
from jax.experimental.pallas import tpu as pltpu

_EPS = 1e-8
_MATCH_THRESH = 0.999
_BB = 128

_NT = (((1,), (1,)), ((), ()))


def _bias_kernel(hypo_ref, prem_ref, w_ref, b_ref, out_ref, *, n_pairs, n_h):
    d = hypo_ref.shape[2]
    inv_h = 1.0 / float(n_h)
    ones_row = jnp.ones((1, d), jnp.float32)
    rows = []
    for i in range(n_pairs):
        h = hypo_ref[i]
        p = prem_ref[i]
        ssq_h = lax.dot_general(h * h, ones_row, _NT,
                                preferred_element_type=jnp.float32)
        ssq_p = lax.dot_general(ones_row, p * p, _NT,
                                preferred_element_type=jnp.float32)
        ih = 1.0 / (jnp.sqrt(ssq_h) + _EPS)
        ip = 1.0 / (jnp.sqrt(ssq_p) + _EPS)
        g = lax.dot_general(h, p, _NT,
                            preferred_element_type=jnp.float32)
        c = g * ih * ip
        mx = jnp.max(c, axis=1, keepdims=True)
        s_mx = jnp.sum(mx, axis=0, keepdims=True)
        mn = jnp.min(mx, axis=0, keepdims=True)
        m = jnp.where(c > _MATCH_THRESH, 1.0, 0.0)
        cnt = jnp.sum(jnp.sum(m, axis=1, keepdims=True), axis=0,
                      keepdims=True)
        f0 = (mn > _MATCH_THRESH).astype(jnp.float32)
        f1 = (cnt == float(n_h)).astype(jnp.float32)
        f2 = cnt * inv_h
        f3 = 1.0 - s_mx * inv_h
        f4 = 1.0 - mn
        rows.append(jnp.concatenate([f0, f1, f2, f3, f4], axis=1))
    feats = jnp.concatenate(rows, axis=0)
    out = lax.dot_general(feats, w_ref[:], _NT,
                          preferred_element_type=jnp.float32)
    out_ref[:] = out + b_ref[:]


@jax.jit
def kernel(hypo, prem, W, b):
    B, H, D = hypo.shape
    P = prem.shape[1]
    bb = _BB
    grid = (B // bb,)
    b2 = b.reshape(1, 3)
    return pl.pallas_call(
        functools.partial(_bias_kernel, n_pairs=bb, n_h=H),
        grid=grid,
        in_specs=[
            pl.BlockSpec((bb, H, D), lambda i: (i, 0, 0)),
            pl.BlockSpec((bb, P, D), lambda i: (i, 0, 0)),
            pl.BlockSpec((3, 5), lambda i: (0, 0)),
            pl.BlockSpec((1, 3), lambda i: (0, 0)),
        ],
        out_specs=pl.BlockSpec((bb, 3), lambda i: (i, 0)),
        out_shape=jax.ShapeDtypeStruct((B, 3), jnp.float32),
        compiler_params=pltpu.CompilerParams(
            dimension_semantics=("parallel",),
        ),
    )(hypo, prem, W, b2)
